# baseline (speedup 1.0000x reference)
import jax
import jax.numpy as jnp
from jax import lax
from jax.experimental import pallas as pl
from jax.experimental.pallas import tpu as pltpu
from jax.experimental.pallas import tpu_sc as plsc

N_USER = 10000
N_ITEM = 40000
N_NODES = N_USER + N_ITEM
D = 64
HALF = 32
N_EDGES = 800000
NC = 2
NS = 16
E_TILE = N_EDGES // NS
SB_E = 512
E_PAD = 50176
N_SB = E_PAD // SB_E
CHUNK = 256
SB_CH = SB_E // CHUNK
N_CHUNK = E_PAD // CHUNK
NBUF = 3
LA = 2
NMETA = 3
DROWS = NS * E_PAD // CHUNK
RCHUNK = 80
N_RCHUNK = N_NODES // RCHUNK
RC_Z = 400
N_RC_Z = N_NODES // RC_Z


def _body(src, dst, vals, ego, zeros_hbm, final_out, e1_out, e2_out,
          acc, srcB, dstB, valB, rows,
          sem_m, sem_g, sem_s):
  c = lax.axis_index("c")
  s = lax.axis_index("s")
  coff = c * N_NODES
  tbase = s * E_PAD
  sbase = (c * NS + s) * E_PAD
  dbase = s * (E_PAD // CHUNK)
  third = jnp.float32(1.0 / 3.0)

  n_rc = (N_RCHUNK // NS) + jnp.where(s < N_RCHUNK % NS, 1, 0)
  n_rcz = (N_RC_Z // NS) + jnp.where(s < N_RC_Z % NS, 1, 0)

  def meta_args(sb, parity):
    o = parity * SB_E
    r0 = dbase + sb * SB_CH
    return (
        (src.at[pl.ds(sbase + sb * SB_E, SB_E)],
         srcB.at[pl.ds(o, SB_E)], sem_m.at[parity]),
        (dst.at[pl.ds(r0, SB_CH)],
         dstB.at[pl.ds(parity * SB_CH, SB_CH)], sem_m.at[parity]),
        (vals.at[pl.ds(tbase + sb * SB_E, SB_E)],
         valB.at[pl.ds(o, SB_E)], sem_m.at[parity]),
    )

  def issue_meta(sb, parity):
    for a in meta_args(sb, parity):
      pltpu.async_copy(*a)

  def wait_meta(sb, parity):
    for a in meta_args(sb, parity):
      pltpu.make_async_copy(*a).wait()

  def layer(tab_in, tab_out, last):
    def zero_body(k, _):
      r0 = (s + k * NS) * RC_Z
      pltpu.sync_copy(zeros_hbm, acc.at[pl.ds(r0, RC_Z)])
      return 0
    lax.fori_loop(0, n_rcz, zero_body, 0)
    plsc.subcore_barrier()

    def _moff(i):
      return ((i // SB_CH) % NMETA) * SB_E + (i % SB_CH) * CHUNK

    def _drow(i):
      return ((i // SB_CH) % NMETA) * SB_CH + (i % SB_CH)

    def stage_a(i):
      b = i % NBUF
      pltpu.async_copy(tab_in.at[srcB.at[pl.ds(_moff(i), CHUNK)]],
                       rows.at[b], sem_g.at[b])

    def wait_gather(i):
      b = i % NBUF
      pltpu.make_async_copy(tab_in.at[srcB.at[pl.ds(_moff(i), CHUNK)]],
                            rows.at[b], sem_g.at[b]).wait()

    def start_scatter(i):
      b = i % NBUF
      pltpu.async_copy(rows.at[b], acc.at[dstB.at[_drow(i)]],
                       sem_s.at[b], add=True)

    def wait_scatter(i):
      b = i % NBUF
      pltpu.make_async_copy(rows.at[b], acc.at[dstB.at[_drow(i)]],
                            sem_s.at[b]).wait()

    def stage_b(i):
      b = i % NBUF
      moff = _moff(i)
      wait_gather(i)
      def scale(g, _):
        v16 = valB[pl.ds(moff + g * 16, 16)]
        for l in range(16):
          e = g * 16 + l
          v = v16[l]
          rows[b, e, pl.ds(0, 16)] = rows[b, e, pl.ds(0, 16)] * v
          rows[b, e, pl.ds(16, 16)] = rows[b, e, pl.ds(16, 16)] * v
        return 0
      lax.fori_loop(0, CHUNK // 16, scale, 0)
      start_scatter(i)

    issue_meta(0, 0)

    def chunk_body(i, _):
      sb = i // SB_CH
      @pl.when(i % SB_CH == 0)
      def _():
        @pl.when(sb + 1 < N_SB)
        def _():
          issue_meta(sb + 1, (sb + 1) % NMETA)
        wait_meta(sb, sb % NMETA)
      @pl.when(i >= NBUF)
      def _():
        wait_scatter(i - NBUF)
      stage_a(i)
      @pl.when(i >= LA)
      def _():
        stage_b(i - LA)
      return 0
    lax.fori_loop(0, N_CHUNK, chunk_body, 0)
    for k in range(LA):
      stage_b(N_CHUNK - LA + k)
    for k in range(NBUF):
      wait_scatter(N_CHUNK - NBUF + k)

    plsc.subcore_barrier()
    if not last:
      def pub_body(k, _):
        r0 = (s + k * NS) * RC_Z
        pltpu.sync_copy(acc.at[pl.ds(r0, RC_Z)],
                        tab_out.at[pl.ds(coff + r0, RC_Z)])
        return 0
      lax.fori_loop(0, n_rcz, pub_body, 0)
      plsc.subcore_barrier()
    else:
      def mean_body(k, _):
        r0 = (s + k * NS) * RCHUNK
        pltpu.sync_copy(e1_out.at[pl.ds(coff + r0, RCHUNK)],
                        rows.at[0, pl.ds(0, RCHUNK)])
        pltpu.sync_copy(e2_out.at[pl.ds(coff + r0, RCHUNK)],
                        rows.at[1, pl.ds(0, RCHUNK)])
        pltpu.sync_copy(acc.at[pl.ds(r0, RCHUNK)],
                        rows.at[2, pl.ds(0, RCHUNK)])
        def mrow(rr, _):
          for j in range(HALF // 16):
            sl = pl.ds(j * 16, 16)
            rows[0, rr, sl] = (rows[0, rr, sl] + rows[1, rr, sl]
                               + rows[2, rr, sl]) * third
          return 0
        lax.fori_loop(0, RCHUNK, mrow, 0)
        pltpu.sync_copy(rows.at[0, pl.ds(0, RCHUNK)],
                        final_out.at[pl.ds(coff + r0, RCHUNK)])
        return 0
      lax.fori_loop(0, n_rc, mean_body, 0)

  layer(ego, e1_out, False)
  layer(e1_out, e2_out, False)
  layer(e2_out, None, True)


@jax.jit
def _run(src, dst, adj_values, ego_half, zeros_hbm):
  mesh = plsc.VectorSubcoreMesh(core_axis_name="c", subcore_axis_name="s")
  f = pl.kernel(
      _body,
      out_type=[
          jax.ShapeDtypeStruct((NC * N_NODES, HALF), jnp.float32),
          jax.ShapeDtypeStruct((NC * N_NODES, HALF), jnp.float32),
          jax.ShapeDtypeStruct((NC * N_NODES, HALF), jnp.float32),
      ],
      mesh=mesh,
      compiler_params=pltpu.CompilerParams(use_tc_tiling_on_sc=False),
      scratch_types=[
          pltpu.VMEM_SHARED((N_NODES, HALF), jnp.float32),
          pltpu.VMEM((NMETA * SB_E,), jnp.int32),
          pltpu.VMEM((NMETA * SB_CH, CHUNK), jnp.int32),
          pltpu.VMEM((NMETA * SB_E,), jnp.float32),
          pltpu.VMEM((NBUF, CHUNK, HALF), jnp.float32),
          pltpu.SemaphoreType.DMA((NMETA,)),
          pltpu.SemaphoreType.DMA((NBUF,)),
          pltpu.SemaphoreType.DMA((NBUF,)),
      ],
  )
  final, _, _ = f(src, dst, adj_values, ego_half, zeros_hbm)
  return final


def _pad_edges(x):
  return jnp.pad(x.reshape(NS, E_TILE), ((0, 0), (0, E_PAD - E_TILE)))


def kernel(adj_indices, adj_values, user_emb, item_emb):
  ego = jnp.concatenate([user_emb, item_emb], axis=0)
  ego_half = jnp.concatenate([ego[:, :HALF], ego[:, HALF:]], axis=0)
  zeros_hbm = jnp.zeros((RC_Z, HALF), jnp.float32)
  src_t = _pad_edges(adj_indices[0])
  srcp = jnp.stack([src_t, src_t + N_NODES]).reshape(-1)
  dstp = _pad_edges(adj_indices[1]).reshape(DROWS, CHUNK)
  valp = _pad_edges(adj_values).reshape(-1)
  half = _run(srcp, dstp, valp, ego_half, zeros_hbm)
  final = jnp.concatenate([half[:N_NODES], half[N_NODES:]], axis=1)
  return (final[:N_USER], final[N_USER:])


# speedup vs baseline: 1.0631x; 1.0631x over previous
import jax
import jax.numpy as jnp
from jax import lax
from jax.experimental import pallas as pl
from jax.experimental.pallas import tpu as pltpu
from jax.experimental.pallas import tpu_sc as plsc

N_USER = 10000
N_ITEM = 40000
N_NODES = N_USER + N_ITEM
D = 64
HALF = 32
N_EDGES = 800000
NC = 2
NS = 16
E_TILE = N_EDGES // NS
SB_E = 512
E_PAD = 50176
N_SB = E_PAD // SB_E
CHUNK = 64
SB_CH = SB_E // CHUNK
N_CHUNK = E_PAD // CHUNK
NBUF = 10
LA = 8
NMETA = 3
DROWS = NS * E_PAD // CHUNK
RCHUNK = 80
N_RCHUNK = N_NODES // RCHUNK
RC_Z = 400
N_RC_Z = N_NODES // RC_Z


def _body(src, dst, vals, ego, zeros_hbm, final_out, e1_out, e2_out,
          acc, srcB, dstB, valB, rows,
          sem_m, sem_g, sem_s):
  c = lax.axis_index("c")
  s = lax.axis_index("s")
  coff = c * N_NODES
  tbase = s * E_PAD
  sbase = (c * NS + s) * E_PAD
  dbase = s * (E_PAD // CHUNK)
  third = jnp.float32(1.0 / 3.0)

  n_rc = (N_RCHUNK // NS) + jnp.where(s < N_RCHUNK % NS, 1, 0)
  n_rcz = (N_RC_Z // NS) + jnp.where(s < N_RC_Z % NS, 1, 0)

  def meta_args(sb, parity):
    o = parity * SB_E
    r0 = dbase + sb * SB_CH
    return (
        (src.at[pl.ds(sbase + sb * SB_E, SB_E)],
         srcB.at[pl.ds(o, SB_E)], sem_m.at[parity]),
        (dst.at[pl.ds(r0, SB_CH)],
         dstB.at[pl.ds(parity * SB_CH, SB_CH)], sem_m.at[parity]),
        (vals.at[pl.ds(tbase + sb * SB_E, SB_E)],
         valB.at[pl.ds(o, SB_E)], sem_m.at[parity]),
    )

  def issue_meta(sb, parity):
    for a in meta_args(sb, parity):
      pltpu.async_copy(*a)

  def wait_meta(sb, parity):
    for a in meta_args(sb, parity):
      pltpu.make_async_copy(*a).wait()

  def layer(tab_in, tab_out, last):
    def zero_body(k, _):
      r0 = (s + k * NS) * RC_Z
      pltpu.sync_copy(zeros_hbm, acc.at[pl.ds(r0, RC_Z)])
      return 0
    lax.fori_loop(0, n_rcz, zero_body, 0)
    plsc.subcore_barrier()

    def _moff(i):
      return ((i // SB_CH) % NMETA) * SB_E + (i % SB_CH) * CHUNK

    def _drow(i):
      return ((i // SB_CH) % NMETA) * SB_CH + (i % SB_CH)

    def stage_a(i):
      b = i % NBUF
      pltpu.async_copy(tab_in.at[srcB.at[pl.ds(_moff(i), CHUNK)]],
                       rows.at[b], sem_g.at[b])

    def wait_gather(i):
      b = i % NBUF
      pltpu.make_async_copy(tab_in.at[srcB.at[pl.ds(_moff(i), CHUNK)]],
                            rows.at[b], sem_g.at[b]).wait()

    def start_scatter(i):
      b = i % NBUF
      pltpu.async_copy(rows.at[b], acc.at[dstB.at[_drow(i)]],
                       sem_s.at[b], add=True)

    def wait_scatter(i):
      b = i % NBUF
      pltpu.make_async_copy(rows.at[b], acc.at[dstB.at[_drow(i)]],
                            sem_s.at[b]).wait()

    def stage_b(i):
      b = i % NBUF
      moff = _moff(i)
      wait_gather(i)
      def scale(g, _):
        v16 = valB[pl.ds(moff + g * 16, 16)]
        for l in range(16):
          e = g * 16 + l
          v = v16[l]
          rows[b, e, pl.ds(0, 16)] = rows[b, e, pl.ds(0, 16)] * v
          rows[b, e, pl.ds(16, 16)] = rows[b, e, pl.ds(16, 16)] * v
        return 0
      lax.fori_loop(0, CHUNK // 16, scale, 0)
      start_scatter(i)

    issue_meta(0, 0)

    def chunk_body(i, _):
      sb = i // SB_CH
      @pl.when(i % SB_CH == 0)
      def _():
        @pl.when(sb + 1 < N_SB)
        def _():
          issue_meta(sb + 1, (sb + 1) % NMETA)
        wait_meta(sb, sb % NMETA)
      @pl.when(i >= NBUF)
      def _():
        wait_scatter(i - NBUF)
      stage_a(i)
      @pl.when(i >= LA)
      def _():
        stage_b(i - LA)
      return 0
    lax.fori_loop(0, N_CHUNK, chunk_body, 0)
    for k in range(LA):
      stage_b(N_CHUNK - LA + k)
    for k in range(NBUF):
      wait_scatter(N_CHUNK - NBUF + k)

    plsc.subcore_barrier()
    if not last:
      def pub_body(k, _):
        r0 = (s + k * NS) * RC_Z
        pltpu.sync_copy(acc.at[pl.ds(r0, RC_Z)],
                        tab_out.at[pl.ds(coff + r0, RC_Z)])
        return 0
      lax.fori_loop(0, n_rcz, pub_body, 0)
      plsc.subcore_barrier()
    else:
      def mean_body(k, _):
        r0 = (s + k * NS) * RCHUNK
        pltpu.sync_copy(e1_out.at[pl.ds(coff + r0, RCHUNK)],
                        rows.at[0, pl.ds(0, RCHUNK)])
        pltpu.sync_copy(e2_out.at[pl.ds(coff + r0, RCHUNK)],
                        rows.at[1, pl.ds(0, RCHUNK)])
        pltpu.sync_copy(acc.at[pl.ds(r0, RCHUNK)],
                        rows.at[2, pl.ds(0, RCHUNK)])
        def mrow(rr, _):
          for j in range(HALF // 16):
            sl = pl.ds(j * 16, 16)
            rows[0, rr, sl] = (rows[0, rr, sl] + rows[1, rr, sl]
                               + rows[2, rr, sl]) * third
          return 0
        lax.fori_loop(0, RCHUNK, mrow, 0)
        pltpu.sync_copy(rows.at[0, pl.ds(0, RCHUNK)],
                        final_out.at[pl.ds(coff + r0, RCHUNK)])
        return 0
      lax.fori_loop(0, n_rc, mean_body, 0)

  layer(ego, e1_out, False)
  layer(e1_out, e2_out, False)
  layer(e2_out, None, True)


@jax.jit
def _run(src, dst, adj_values, ego_half, zeros_hbm):
  mesh = plsc.VectorSubcoreMesh(core_axis_name="c", subcore_axis_name="s")
  f = pl.kernel(
      _body,
      out_type=[
          jax.ShapeDtypeStruct((NC * N_NODES, HALF), jnp.float32),
          jax.ShapeDtypeStruct((NC * N_NODES, HALF), jnp.float32),
          jax.ShapeDtypeStruct((NC * N_NODES, HALF), jnp.float32),
      ],
      mesh=mesh,
      compiler_params=pltpu.CompilerParams(use_tc_tiling_on_sc=False),
      scratch_types=[
          pltpu.VMEM_SHARED((N_NODES, HALF), jnp.float32),
          pltpu.VMEM((NMETA * SB_E,), jnp.int32),
          pltpu.VMEM((NMETA * SB_CH, CHUNK), jnp.int32),
          pltpu.VMEM((NMETA * SB_E,), jnp.float32),
          pltpu.VMEM((NBUF, CHUNK, HALF), jnp.float32),
          pltpu.SemaphoreType.DMA((NMETA,)),
          pltpu.SemaphoreType.DMA((NBUF,)),
          pltpu.SemaphoreType.DMA((NBUF,)),
      ],
  )
  final, _, _ = f(src, dst, adj_values, ego_half, zeros_hbm)
  return final


def _pad_edges(x):
  return jnp.pad(x.reshape(NS, E_TILE), ((0, 0), (0, E_PAD - E_TILE)))


def kernel(adj_indices, adj_values, user_emb, item_emb):
  ego = jnp.concatenate([user_emb, item_emb], axis=0)
  ego_half = jnp.concatenate([ego[:, :HALF], ego[:, HALF:]], axis=0)
  zeros_hbm = jnp.zeros((RC_Z, HALF), jnp.float32)
  src_t = _pad_edges(adj_indices[0])
  srcp = jnp.stack([src_t, src_t + N_NODES]).reshape(-1)
  dstp = _pad_edges(adj_indices[1]).reshape(DROWS, CHUNK)
  valp = _pad_edges(adj_values).reshape(-1)
  half = _run(srcp, dstp, valp, ego_half, zeros_hbm)
  final = jnp.concatenate([half[:N_NODES], half[N_NODES:]], axis=1)
  return (final[:N_USER], final[N_USER:])


# speedup vs baseline: 1.1073x; 1.0416x over previous
import jax
import jax.numpy as jnp
from jax import lax
from jax.experimental import pallas as pl
from jax.experimental.pallas import tpu as pltpu
from jax.experimental.pallas import tpu_sc as plsc

N_USER = 10000
N_ITEM = 40000
N_NODES = N_USER + N_ITEM
D = 64
HALF = 32
N_EDGES = 800000
NC = 2
NS = 16
E_TILE = N_EDGES // NS
SB_E = 512
E_PAD = 50176
N_SB = E_PAD // SB_E
CHUNK = 128
SB_CH = SB_E // CHUNK
N_CHUNK = E_PAD // CHUNK
NBUF = 6
LA = 4
NMETA = 3
DROWS = NS * E_PAD // CHUNK
RCHUNK = 80
N_RCHUNK = N_NODES // RCHUNK
RC_Z = 400
N_RC_Z = N_NODES // RC_Z


def _body(src, dst, vals, ego, zeros_hbm, final_out, e1_out, e2_out,
          acc, srcB, dstB, valB, rows,
          sem_m, sem_g, sem_s):
  c = lax.axis_index("c")
  s = lax.axis_index("s")
  coff = c * N_NODES
  tbase = s * E_PAD
  sbase = (c * NS + s) * E_PAD
  dbase = s * (E_PAD // CHUNK)
  third = jnp.float32(1.0 / 3.0)

  n_rc = (N_RCHUNK // NS) + jnp.where(s < N_RCHUNK % NS, 1, 0)
  n_rcz = (N_RC_Z // NS) + jnp.where(s < N_RC_Z % NS, 1, 0)

  def meta_args(sb, parity):
    o = parity * SB_E
    r0 = dbase + sb * SB_CH
    return (
        (src.at[pl.ds(sbase + sb * SB_E, SB_E)],
         srcB.at[pl.ds(o, SB_E)], sem_m.at[parity]),
        (dst.at[pl.ds(r0, SB_CH)],
         dstB.at[pl.ds(parity * SB_CH, SB_CH)], sem_m.at[parity]),
        (vals.at[pl.ds(tbase + sb * SB_E, SB_E)],
         valB.at[pl.ds(o, SB_E)], sem_m.at[parity]),
    )

  def issue_meta(sb, parity):
    for a in meta_args(sb, parity):
      pltpu.async_copy(*a)

  def wait_meta(sb, parity):
    for a in meta_args(sb, parity):
      pltpu.make_async_copy(*a).wait()

  def layer(tab_in, tab_out, last):
    def zero_issue(k, _):
      r0 = (s + k * NS) * RC_Z
      pltpu.async_copy(zeros_hbm, acc.at[pl.ds(r0, RC_Z)], sem_m.at[0])
      return 0
    lax.fori_loop(0, n_rcz, zero_issue, 0)
    def zero_wait(k, _):
      r0 = (s + k * NS) * RC_Z
      pltpu.make_async_copy(zeros_hbm, acc.at[pl.ds(r0, RC_Z)],
                            sem_m.at[0]).wait()
      return 0
    lax.fori_loop(0, n_rcz, zero_wait, 0)
    plsc.subcore_barrier()

    def _moff(i):
      return ((i // SB_CH) % NMETA) * SB_E + (i % SB_CH) * CHUNK

    def _drow(i):
      return ((i // SB_CH) % NMETA) * SB_CH + (i % SB_CH)

    def stage_a(i):
      b = i % NBUF
      pltpu.async_copy(tab_in.at[srcB.at[pl.ds(_moff(i), CHUNK)]],
                       rows.at[b], sem_g.at[b])

    def wait_gather(i):
      b = i % NBUF
      pltpu.make_async_copy(tab_in.at[srcB.at[pl.ds(_moff(i), CHUNK)]],
                            rows.at[b], sem_g.at[b]).wait()

    def start_scatter(i):
      b = i % NBUF
      pltpu.async_copy(rows.at[b], acc.at[dstB.at[_drow(i)]],
                       sem_s.at[b], add=True)

    def wait_scatter(i):
      b = i % NBUF
      pltpu.make_async_copy(rows.at[b], acc.at[dstB.at[_drow(i)]],
                            sem_s.at[b]).wait()

    def stage_b(i):
      b = i % NBUF
      moff = _moff(i)
      wait_gather(i)
      def scale(g, _):
        v16 = valB[pl.ds(moff + g * 16, 16)]
        for l in range(16):
          e = g * 16 + l
          v = v16[l]
          rows[b, e, pl.ds(0, 16)] = rows[b, e, pl.ds(0, 16)] * v
          rows[b, e, pl.ds(16, 16)] = rows[b, e, pl.ds(16, 16)] * v
        return 0
      lax.fori_loop(0, CHUNK // 16, scale, 0)
      start_scatter(i)

    issue_meta(0, 0)

    def chunk_body(i, _):
      sb = i // SB_CH
      @pl.when(i % SB_CH == 0)
      def _():
        @pl.when(sb + 1 < N_SB)
        def _():
          issue_meta(sb + 1, (sb + 1) % NMETA)
        wait_meta(sb, sb % NMETA)
      @pl.when(i >= NBUF)
      def _():
        wait_scatter(i - NBUF)
      stage_a(i)
      @pl.when(i >= LA)
      def _():
        stage_b(i - LA)
      return 0
    lax.fori_loop(0, N_CHUNK, chunk_body, 0)
    for k in range(LA):
      stage_b(N_CHUNK - LA + k)
    for k in range(NBUF):
      wait_scatter(N_CHUNK - NBUF + k)

    plsc.subcore_barrier()
    if not last:
      def pub_issue(k, _):
        r0 = (s + k * NS) * RC_Z
        pltpu.async_copy(acc.at[pl.ds(r0, RC_Z)],
                         tab_out.at[pl.ds(coff + r0, RC_Z)], sem_m.at[1])
        return 0
      lax.fori_loop(0, n_rcz, pub_issue, 0)
      def pub_wait(k, _):
        r0 = (s + k * NS) * RC_Z
        pltpu.make_async_copy(acc.at[pl.ds(r0, RC_Z)],
                              tab_out.at[pl.ds(coff + r0, RC_Z)],
                              sem_m.at[1]).wait()
        return 0
      lax.fori_loop(0, n_rcz, pub_wait, 0)
      plsc.subcore_barrier()
    else:
      def m_load(k, slot):
        r0 = (s + k * NS) * RCHUNK
        pltpu.async_copy(e1_out.at[pl.ds(coff + r0, RCHUNK)],
                         rows.at[slot, pl.ds(0, RCHUNK)], sem_g.at[slot])
        pltpu.async_copy(e2_out.at[pl.ds(coff + r0, RCHUNK)],
                         rows.at[slot + 1, pl.ds(0, RCHUNK)],
                         sem_g.at[slot + 1])
        pltpu.async_copy(acc.at[pl.ds(r0, RCHUNK)],
                         rows.at[slot + 2, pl.ds(0, RCHUNK)],
                         sem_g.at[slot + 2])

      def m_wait_load(k, slot):
        r0 = (s + k * NS) * RCHUNK
        pltpu.make_async_copy(e1_out.at[pl.ds(coff + r0, RCHUNK)],
                              rows.at[slot, pl.ds(0, RCHUNK)],
                              sem_g.at[slot]).wait()
        pltpu.make_async_copy(e2_out.at[pl.ds(coff + r0, RCHUNK)],
                              rows.at[slot + 1, pl.ds(0, RCHUNK)],
                              sem_g.at[slot + 1]).wait()
        pltpu.make_async_copy(acc.at[pl.ds(r0, RCHUNK)],
                              rows.at[slot + 2, pl.ds(0, RCHUNK)],
                              sem_g.at[slot + 2]).wait()

      def m_store(k, slot):
        r0 = (s + k * NS) * RCHUNK
        pltpu.async_copy(rows.at[slot, pl.ds(0, RCHUNK)],
                         final_out.at[pl.ds(coff + r0, RCHUNK)],
                         sem_s.at[slot])

      def m_wait_store(k, slot):
        r0 = (s + k * NS) * RCHUNK
        pltpu.make_async_copy(rows.at[slot, pl.ds(0, RCHUNK)],
                              final_out.at[pl.ds(coff + r0, RCHUNK)],
                              sem_s.at[slot]).wait()

      @pl.when(n_rc >= 1)
      def _():
        m_load(0, 0)
      def mean_body(k, _):
        slot = 3 * (k % 2)
        nslot = 3 - slot
        @pl.when(k + 1 < n_rc)
        def _():
          @pl.when(k >= 1)
          def _():
            m_wait_store(k - 1, nslot)
          m_load(k + 1, nslot)
        m_wait_load(k, slot)
        def mrow(rr, _):
          for j in range(HALF // 16):
            sl = pl.ds(j * 16, 16)
            rows[slot, rr, sl] = (rows[slot, rr, sl] + rows[slot + 1, rr, sl]
                                  + rows[slot + 2, rr, sl]) * third
          return 0
        lax.fori_loop(0, RCHUNK, mrow, 0)
        m_store(k, slot)
        return 0
      lax.fori_loop(0, n_rc, mean_body, 0)
      @pl.when(n_rc >= 2)
      def _():
        m_wait_store(n_rc - 2, 3 * ((n_rc - 2) % 2))
      @pl.when(n_rc >= 1)
      def _():
        m_wait_store(n_rc - 1, 3 * ((n_rc - 1) % 2))

  layer(ego, e1_out, False)
  layer(e1_out, e2_out, False)
  layer(e2_out, None, True)


@jax.jit
def _run(src, dst, adj_values, ego_half, zeros_hbm):
  mesh = plsc.VectorSubcoreMesh(core_axis_name="c", subcore_axis_name="s")
  f = pl.kernel(
      _body,
      out_type=[
          jax.ShapeDtypeStruct((NC * N_NODES, HALF), jnp.float32),
          jax.ShapeDtypeStruct((NC * N_NODES, HALF), jnp.float32),
          jax.ShapeDtypeStruct((NC * N_NODES, HALF), jnp.float32),
      ],
      mesh=mesh,
      compiler_params=pltpu.CompilerParams(use_tc_tiling_on_sc=False),
      scratch_types=[
          pltpu.VMEM_SHARED((N_NODES, HALF), jnp.float32),
          pltpu.VMEM((NMETA * SB_E,), jnp.int32),
          pltpu.VMEM((NMETA * SB_CH, CHUNK), jnp.int32),
          pltpu.VMEM((NMETA * SB_E,), jnp.float32),
          pltpu.VMEM((NBUF, CHUNK, HALF), jnp.float32),
          pltpu.SemaphoreType.DMA((NMETA,)),
          pltpu.SemaphoreType.DMA((NBUF,)),
          pltpu.SemaphoreType.DMA((NBUF,)),
      ],
  )
  final, _, _ = f(src, dst, adj_values, ego_half, zeros_hbm)
  return final


def _pad_edges(x):
  return jnp.pad(x.reshape(NS, E_TILE), ((0, 0), (0, E_PAD - E_TILE)))


def kernel(adj_indices, adj_values, user_emb, item_emb):
  ego = jnp.concatenate([user_emb, item_emb], axis=0)
  ego_half = jnp.concatenate([ego[:, :HALF], ego[:, HALF:]], axis=0)
  zeros_hbm = jnp.zeros((RC_Z, HALF), jnp.float32)
  src_t = _pad_edges(adj_indices[0])
  srcp = jnp.stack([src_t, src_t + N_NODES]).reshape(-1)
  dstp = _pad_edges(adj_indices[1]).reshape(DROWS, CHUNK)
  valp = _pad_edges(adj_values).reshape(-1)
  half = _run(srcp, dstp, valp, ego_half, zeros_hbm)
  final = jnp.concatenate([half[:N_NODES], half[N_NODES:]], axis=1)
  return (final[:N_USER], final[N_USER:])
